# baseline (speedup 1.0000x reference)
import functools

import jax
import jax.numpy as jnp
from jax import lax
from jax.experimental import pallas as pl
from jax.experimental.pallas import tpu as pltpu
from jax.experimental.pallas import tpu_sc as plsc

_EPS = 1e-7
_N = 8192
_F = 10
_BM = 512
_NTS = _N // _BM
_NC = 1
_NS = 16
_NW = _NC * _NS
_CHUNK = _N // _NW
_NVREG = _CHUNK // 16
_WIN = _CHUNK + 16


def _sc_compact_kernel(ti_ref, ph_ref, out_ref, ti_v, lidx_v, rows_v, sem):
    c = lax.axis_index("c")
    s = lax.axis_index("s")
    wid = s * _NC + c
    base_vreg = wid * _NVREG
    pltpu.sync_copy(ti_ref, ti_v)

    zero = jnp.zeros((16,), jnp.int32)
    for j in range(_WIN // 16):
        lidx_v[pl.ds(j * 16, 16)] = zero

    zero_splat = jnp.zeros((16,), jnp.int32)
    lanes = lax.iota(jnp.int32, 16)

    def pre(g, cnt):
        for u in range(4):
            j = g * 4 + u
            v = ti_v[pl.ds(j * 16, 16)]
            pc = plsc.all_reduce_population_count(v == 1)
            cnt = cnt + jnp.where(j < base_vreg, pc, zero_splat)
        return cnt

    off = lax.fori_loop(0, _N // 64, pre, zero_splat)
    wstart = off - lax.rem(off, 8)

    def body(g, cnt):
        for u in range(4):
            j = g * 4 + u
            v = ti_v[pl.ds(j * 16, 16)]
            m = v == 1
            mi = jnp.where(m, 1, 0).astype(jnp.int32)
            gpos = cnt + plsc.cumsum(mi) - 1
            slot = gpos - wstart
            valid = m & (slot >= 0) & (slot < _WIN)
            slot = jnp.where(valid, slot, 0)
            gidx = j * 16 + lanes
            plsc.store_scatter(lidx_v, [slot], gidx, mask=valid)
            cnt = cnt + plsc.all_reduce_population_count(m)
        return cnt

    lax.fori_loop(0, _N // 64, body, zero_splat)

    pltpu.async_copy(ph_ref.at[lidx_v], rows_v, sem).wait()
    wstart_s = jnp.sum(jnp.where(lanes == 0, wstart, zero_splat))
    pltpu.sync_copy(rows_v,
                    out_ref.at[pl.ds(pl.multiple_of(wstart_s, 8), _WIN)])


_sc_compact_cache = []


def _sc_compact(ti, ph128):
    if not _sc_compact_cache:
        _sc_compact_cache.append(functools.partial(
            pl.kernel,
            mesh=plsc.VectorSubcoreMesh(
                core_axis_name="c", subcore_axis_name="s", num_cores=_NC),
            compiler_params=pltpu.CompilerParams(needs_layout_passes=False),
            out_type=jax.ShapeDtypeStruct((_N + 16, 128), jnp.float32),
            scratch_types=[
                pltpu.VMEM((_N,), jnp.int32),
                pltpu.VMEM((_WIN,), jnp.int32),
                pltpu.VMEM((_WIN, 128), jnp.float32),
                pltpu.SemaphoreType.DMA,
            ],
        )(_sc_compact_kernel))
    return _sc_compact_cache[0](ti, ph128)


def _tc_kernel(phT_ref, ti_row_ref, estT_flatT_ref, gtT_flatT_ref,
               psel_ref, ep1_ref, gp1_ref,
               cham_ref, l2_ref,
               gsel_scr, esel_scr, acc1, acc2):
    phT = phT_ref[...]
    ti_row = ti_row_ref[...]
    estT_flatT = estT_flatT_ref[...]
    gtT_flatT = gtT_flatT_ref[...]

    iota_row = lax.broadcasted_iota(jnp.int32, (_F, _N), 0)
    onehot_rowT = (iota_row == ti_row).astype(jnp.float32)
    BeT = jnp.dot(estT_flatT, onehot_rowT,
                  preferred_element_type=jnp.float32,
                  precision=lax.Precision.HIGHEST)
    BgT = jnp.dot(gtT_flatT, onehot_rowT,
                  preferred_element_type=jnp.float32,
                  precision=lax.Precision.HIGHEST)

    def rows_from(BT):
        rs = []
        for k in range(3):
            acc = BT[4 * k:4 * k + 1, :] * phT[0:1, :]
            for j in range(1, 4):
                acc = acc + BT[4 * k + j:4 * k + j + 1, :] * phT[j:j + 1, :]
            rs.append(acc)
        return jnp.concatenate(rs, axis=0)

    estT = rows_from(BeT)
    gtT = rows_from(BgT)
    en2_row = jnp.sum(estT * estT, axis=0, keepdims=True)
    gn2_row = jnp.sum(gtT * gtT, axis=0, keepdims=True)

    w_row_i = jnp.where(ti_row == 1, 1, 0).astype(jnp.int32)
    w_row = w_row_i.astype(jnp.float32)
    cnt = jnp.sum(w_row)
    m = jnp.sum(w_row_i)

    x = psel_ref[:, 0:1]
    y = psel_ref[:, 1:2]
    z = psel_ref[:, 2:3]

    def tsfm(p_ref):
        cols = []
        for k in range(3):
            acc = p_ref[4 * k] * x
            acc = acc + p_ref[4 * k + 1] * y
            acc = acc + p_ref[4 * k + 2] * z
            acc = acc + p_ref[4 * k + 3]
            cols.append(acc)
        return cols

    ex, ey, ez = tsfm(ep1_ref)
    gx, gy, gz = tsfm(gp1_ref)
    en2_sel = ex * ex + ey * ey + ez * ez
    gn2_sel = gx * gx + gy * gy + gz * gz
    gsel_scr[...] = jnp.concatenate(
        [-2.0 * gx, -2.0 * gy, -2.0 * gz], axis=1)
    esel_scr[...] = jnp.concatenate(
        [-2.0 * ex, -2.0 * ey, -2.0 * ez], axis=1)

    acc1[...] = jnp.zeros((_BM, 1), jnp.float32)
    acc2[...] = jnp.zeros((_BM, 1), jnp.float32)

    for g in range(_NTS):
        start = g * _BM

        @pl.when(start < m)
        def _side1():
            Gt = gsel_scr[start:start + _BM, :]
            cross = lax.dot_general(
                Gt, estT, (((1,), (0,)), ((), ())),
                preferred_element_type=jnp.float32,
                precision=lax.Precision.DEFAULT)
            rmin = jnp.min(cross + en2_row, axis=1, keepdims=True)
            rid = lax.broadcasted_iota(jnp.int32, (_BM, 1), 0) + start
            acc1[...] += jnp.where(rid < m, rmin, 0.0)

        @pl.when(start < m)
        def _side2():
            Et = esel_scr[start:start + _BM, :]
            cross = lax.dot_general(
                Et, gtT, (((1,), (0,)), ((), ())),
                preferred_element_type=jnp.float32,
                precision=lax.Precision.DEFAULT)
            rmin = jnp.min(cross + gn2_row, axis=1, keepdims=True)
            rid = lax.broadcasted_iota(jnp.int32, (_BM, 1), 0) + start
            acc2[...] += jnp.where(rid < m, rmin, 0.0)

    sel_id = lax.broadcasted_iota(jnp.int32, (_N, 1), 0)
    sel_mask = sel_id < m
    s1 = jnp.sum(acc1[...]) + jnp.sum(jnp.where(sel_mask, gn2_sel, 0.0))
    s2 = jnp.sum(acc2[...]) + jnp.sum(jnp.where(sel_mask, en2_sel, 0.0))

    denom = cnt + _EPS
    cham_ref[0, 0] = ((s1 + s2) / denom) * 0.5

    diffT = estT - gtT
    nrm = jnp.sqrt(jnp.sum(diffT * diffT, axis=0, keepdims=True))
    l2_ref[0, 0] = jnp.sum(w_row * nrm) / denom


def kernel(points, time_indice, est_poses, gt_poses):
    ph = jnp.concatenate(
        [points, jnp.ones((_N, 1), points.dtype)], axis=1)
    phT = ph.T
    ti_row = time_indice.reshape(1, _N)
    estT_flatT = est_poses[:, :3, :4].reshape(_F, 12).T
    gtT_flatT = gt_poses[:, :3, :4].reshape(_F, 12).T
    ep1 = est_poses[1, :3, :4].reshape(12)
    gp1 = gt_poses[1, :3, :4].reshape(12)

    ph128 = jnp.concatenate(
        [points, jnp.zeros((_N, 125), points.dtype)], axis=1)
    staged = _sc_compact(time_indice, ph128)[:_N]

    cham, l2 = pl.pallas_call(
        _tc_kernel,
        out_shape=(
            jax.ShapeDtypeStruct((1, 1), jnp.float32),
            jax.ShapeDtypeStruct((1, 1), jnp.float32),
        ),
        in_specs=[
            pl.BlockSpec((4, _N), lambda: (0, 0)),
            pl.BlockSpec((1, _N), lambda: (0, 0)),
            pl.BlockSpec((12, _F), lambda: (0, 0)),
            pl.BlockSpec((12, _F), lambda: (0, 0)),
            pl.BlockSpec((_N, 128), lambda: (0, 0)),
            pl.BlockSpec(memory_space=pltpu.SMEM),
            pl.BlockSpec(memory_space=pltpu.SMEM),
        ],
        out_specs=(
            pl.BlockSpec(memory_space=pltpu.SMEM),
            pl.BlockSpec(memory_space=pltpu.SMEM),
        ),
        scratch_shapes=[
            pltpu.VMEM((_N, 3), jnp.float32),
            pltpu.VMEM((_N, 3), jnp.float32),
            pltpu.VMEM((_BM, 1), jnp.float32),
            pltpu.VMEM((_BM, 1), jnp.float32),
        ],
    )(phT, ti_row, estT_flatT, gtT_flatT, staged, ep1, gp1)
    return cham[0, 0], l2[0, 0]


# speedup vs baseline: 1.2331x; 1.2331x over previous
import functools

import jax
import jax.numpy as jnp
from jax import lax
from jax.experimental import pallas as pl
from jax.experimental.pallas import tpu as pltpu
from jax.experimental.pallas import tpu_sc as plsc

_EPS = 1e-7
_N = 8192
_F = 10
_BM = 512
_NTS = _N // _BM
_NC = 2
_NS = 16
_NW = _NC * _NS
_CHUNK = _N // _NW
_NVREG = _CHUNK // 16
_WIN = _CHUNK + 16


def _sc_compact_kernel(ti_ref, ph_ref, out_ref, ti_v, lidx_v, rows_v, sem):
    c = lax.axis_index("c")
    s = lax.axis_index("s")
    wid = s * _NC + c
    base_vreg = wid * _NVREG
    pltpu.sync_copy(ti_ref, ti_v)

    zero = jnp.zeros((16,), jnp.int32)
    for j in range(_WIN // 16):
        lidx_v[pl.ds(j * 16, 16)] = zero

    zero_splat = jnp.zeros((16,), jnp.int32)
    lanes = lax.iota(jnp.int32, 16)

    def pre(g, cnt):
        for u in range(4):
            j = g * 4 + u
            v = ti_v[pl.ds(j * 16, 16)]
            pc = plsc.all_reduce_population_count(v == 1)
            cnt = cnt + jnp.where(j < base_vreg, pc, zero_splat)
        return cnt

    off = lax.fori_loop(0, _N // 64, pre, zero_splat)
    wstart = off - lax.rem(off, 8)

    def body(g, cnt):
        for u in range(4):
            j = g * 4 + u
            v = ti_v[pl.ds(j * 16, 16)]
            m = v == 1
            mi = jnp.where(m, 1, 0).astype(jnp.int32)
            gpos = cnt + plsc.cumsum(mi) - 1
            slot = gpos - wstart
            valid = m & (slot >= 0) & (slot < _WIN)
            slot = jnp.where(valid, slot, 0)
            gidx = j * 16 + lanes
            plsc.store_scatter(lidx_v, [slot], gidx, mask=valid)
            cnt = cnt + plsc.all_reduce_population_count(m)
        return cnt

    lax.fori_loop(0, _N // 64, body, zero_splat)

    pltpu.async_copy(ph_ref.at[lidx_v], rows_v, sem).wait()
    wstart_s = jnp.sum(jnp.where(lanes == 0, wstart, zero_splat))
    pltpu.sync_copy(rows_v,
                    out_ref.at[pl.ds(pl.multiple_of(wstart_s, 8), _WIN)])


_sc_compact_cache = []


def _sc_compact(ti, ph128):
    if not _sc_compact_cache:
        _sc_compact_cache.append(functools.partial(
            pl.kernel,
            mesh=plsc.VectorSubcoreMesh(
                core_axis_name="c", subcore_axis_name="s", num_cores=_NC),
            compiler_params=pltpu.CompilerParams(needs_layout_passes=False),
            out_type=jax.ShapeDtypeStruct((_N + 16, 128), jnp.float32),
            scratch_types=[
                pltpu.VMEM((_N,), jnp.int32),
                pltpu.VMEM((_WIN,), jnp.int32),
                pltpu.VMEM((_WIN, 128), jnp.float32),
                pltpu.SemaphoreType.DMA,
            ],
        )(_sc_compact_kernel))
    return _sc_compact_cache[0](ti, ph128)


def _tc_kernel(phT_ref, ti_row_ref, estT_flatT_ref, gtT_flatT_ref,
               psel_ref, ep1_ref, gp1_ref,
               cham_ref, l2_ref,
               gsel_scr, esel_scr, acc1, acc2):
    phT = phT_ref[...]
    ti_row = ti_row_ref[...]
    estT_flatT = estT_flatT_ref[...]
    gtT_flatT = gtT_flatT_ref[...]

    iota_row = lax.broadcasted_iota(jnp.int32, (_F, _N), 0)
    onehot_rowT = (iota_row == ti_row).astype(jnp.float32)
    BeT = jnp.dot(estT_flatT, onehot_rowT,
                  preferred_element_type=jnp.float32,
                  precision=lax.Precision.HIGHEST)
    BgT = jnp.dot(gtT_flatT, onehot_rowT,
                  preferred_element_type=jnp.float32,
                  precision=lax.Precision.HIGHEST)

    def rows_from(BT):
        rs = []
        for k in range(3):
            acc = BT[4 * k:4 * k + 1, :] * phT[0:1, :]
            for j in range(1, 4):
                acc = acc + BT[4 * k + j:4 * k + j + 1, :] * phT[j:j + 1, :]
            rs.append(acc)
        return jnp.concatenate(rs, axis=0)

    estT = rows_from(BeT)
    gtT = rows_from(BgT)
    en2_row = jnp.sum(estT * estT, axis=0, keepdims=True)
    gn2_row = jnp.sum(gtT * gtT, axis=0, keepdims=True)

    w_row_i = jnp.where(ti_row == 1, 1, 0).astype(jnp.int32)
    w_row = w_row_i.astype(jnp.float32)
    cnt = jnp.sum(w_row)
    m = jnp.sum(w_row_i)

    x = psel_ref[:, 0:1]
    y = psel_ref[:, 1:2]
    z = psel_ref[:, 2:3]

    def tsfm(p_ref):
        cols = []
        for k in range(3):
            acc = p_ref[4 * k] * x
            acc = acc + p_ref[4 * k + 1] * y
            acc = acc + p_ref[4 * k + 2] * z
            acc = acc + p_ref[4 * k + 3]
            cols.append(acc)
        return cols

    ex, ey, ez = tsfm(ep1_ref)
    gx, gy, gz = tsfm(gp1_ref)
    en2_sel = ex * ex + ey * ey + ez * ez
    gn2_sel = gx * gx + gy * gy + gz * gz
    gsel_scr[...] = jnp.concatenate(
        [-2.0 * gx, -2.0 * gy, -2.0 * gz], axis=1)
    esel_scr[...] = jnp.concatenate(
        [-2.0 * ex, -2.0 * ey, -2.0 * ez], axis=1)

    acc1[...] = jnp.zeros((_BM, 1), jnp.float32)
    acc2[...] = jnp.zeros((_BM, 1), jnp.float32)

    for g in range(_NTS):
        start = g * _BM

        @pl.when(start < m)
        def _side1():
            Gt = gsel_scr[start:start + _BM, :]
            cross = lax.dot_general(
                Gt, estT, (((1,), (0,)), ((), ())),
                preferred_element_type=jnp.float32,
                precision=lax.Precision.DEFAULT)
            rmin = jnp.min(cross + en2_row, axis=1, keepdims=True)
            rid = lax.broadcasted_iota(jnp.int32, (_BM, 1), 0) + start
            acc1[...] += jnp.where(rid < m, rmin, 0.0)

        @pl.when(start < m)
        def _side2():
            Et = esel_scr[start:start + _BM, :]
            cross = lax.dot_general(
                Et, gtT, (((1,), (0,)), ((), ())),
                preferred_element_type=jnp.float32,
                precision=lax.Precision.DEFAULT)
            rmin = jnp.min(cross + gn2_row, axis=1, keepdims=True)
            rid = lax.broadcasted_iota(jnp.int32, (_BM, 1), 0) + start
            acc2[...] += jnp.where(rid < m, rmin, 0.0)

    sel_id = lax.broadcasted_iota(jnp.int32, (_N, 1), 0)
    sel_mask = sel_id < m
    s1 = jnp.sum(acc1[...]) + jnp.sum(jnp.where(sel_mask, gn2_sel, 0.0))
    s2 = jnp.sum(acc2[...]) + jnp.sum(jnp.where(sel_mask, en2_sel, 0.0))

    denom = cnt + _EPS
    cham_ref[0, 0] = ((s1 + s2) / denom) * 0.5

    diffT = estT - gtT
    nrm = jnp.sqrt(jnp.sum(diffT * diffT, axis=0, keepdims=True))
    l2_ref[0, 0] = jnp.sum(w_row * nrm) / denom


def kernel(points, time_indice, est_poses, gt_poses):
    ph = jnp.concatenate(
        [points, jnp.ones((_N, 1), points.dtype)], axis=1)
    phT = ph.T
    ti_row = time_indice.reshape(1, _N)
    estT_flatT = est_poses[:, :3, :4].reshape(_F, 12).T
    gtT_flatT = gt_poses[:, :3, :4].reshape(_F, 12).T
    ep1 = est_poses[1, :3, :4].reshape(12)
    gp1 = gt_poses[1, :3, :4].reshape(12)

    ph128 = jnp.concatenate(
        [points, jnp.zeros((_N, 125), points.dtype)], axis=1)
    staged = _sc_compact(time_indice, ph128)[:_N]

    cham, l2 = pl.pallas_call(
        _tc_kernel,
        out_shape=(
            jax.ShapeDtypeStruct((1, 1), jnp.float32),
            jax.ShapeDtypeStruct((1, 1), jnp.float32),
        ),
        in_specs=[
            pl.BlockSpec((4, _N), lambda: (0, 0)),
            pl.BlockSpec((1, _N), lambda: (0, 0)),
            pl.BlockSpec((12, _F), lambda: (0, 0)),
            pl.BlockSpec((12, _F), lambda: (0, 0)),
            pl.BlockSpec((_N, 128), lambda: (0, 0)),
            pl.BlockSpec(memory_space=pltpu.SMEM),
            pl.BlockSpec(memory_space=pltpu.SMEM),
        ],
        out_specs=(
            pl.BlockSpec(memory_space=pltpu.SMEM),
            pl.BlockSpec(memory_space=pltpu.SMEM),
        ),
        scratch_shapes=[
            pltpu.VMEM((_N, 3), jnp.float32),
            pltpu.VMEM((_N, 3), jnp.float32),
            pltpu.VMEM((_BM, 1), jnp.float32),
            pltpu.VMEM((_BM, 1), jnp.float32),
        ],
    )(phT, ti_row, estT_flatT, gtT_flatT, staged, ep1, gp1)
    return cham[0, 0], l2[0, 0]
